# Initial kernel scaffold; baseline (speedup 1.0000x reference)
import jax
import jax.numpy as jnp
from jax.experimental import pallas as pl


def kernel(x, edge_index, W1, a1s, a1d, b1, g1, be1, W2, b2, g2, be2, Wl3, bl3, Wr3, g3, be3, W4, a4s, a4d, b4, Wres, bres):
    raise NotImplementedError("write your pallas kernel here")


# speedup vs baseline: 3.2013x; 3.2013x over previous
import functools

import jax
import jax.numpy as jnp
from jax import lax
from jax.experimental import pallas as pl
from jax.experimental.pallas import tpu as pltpu
from jax.experimental.pallas import tpu_sc as plsc

N = 10000
E = 160000
F_IN = 256
H = 512
C = 128

CHUNK = 80
NCHUNK = 128
NP = CHUNK * NCHUNK
KT = 32
NWORK = 32
CPW = NCHUNK // NWORK


def _f32(shape):
    return jax.ShapeDtypeStruct(shape, jnp.float32)


def _make_spmm(D, mode):
    n_tab = {"gat": 2, "gcn": 1, "sage": 0}[mode]
    _mesh = plsc.VectorSubcoreMesh(core_axis_name="c", subcore_axis_name="s")

    scratch = [
        pltpu.VMEM((CHUNK, D), jnp.float32),
        pltpu.VMEM((CHUNK, 16), jnp.float32),
        pltpu.VMEM((NCHUNK + 16,), jnp.int32),
        pltpu.VMEM((KT,), jnp.int32),
        pltpu.VMEM((KT + 16,), jnp.int32),
        pltpu.VMEM((KT + 16,), jnp.float32),
        pltpu.VMEM((KT, D), jnp.float32),
        pltpu.SemaphoreType.DMA,
    ] + [pltpu.VMEM((NP,), jnp.float32)] * n_tab

    out_type = [_f32((NP, D))] + ([_f32((NP, 16))] if mode == "gat" else [])

    @functools.partial(
        pl.kernel, out_type=out_type, mesh=_mesh, scratch_types=scratch,
        compiler_params=pltpu.CompilerParams(needs_layout_passes=False))
    def spmm(tab_hbm, srcs_hbm, dsts_hbm, cb_hbm, *rest):
        tabs_hbm = rest[:n_tab]
        if mode == "gat":
            m_hbm, s_hbm = rest[n_tab:n_tab + 2]
            rest = rest[n_tab + 2:]
        else:
            m_hbm = rest[n_tab]
            rest = rest[n_tab + 1:]
        acc_m, acc_s, cb_v, idx_v, dst_v, w_v, rows_v, sem = rest[:8]
        tabs_v = rest[8:8 + n_tab]

        wid = lax.axis_index("s") * 2 + lax.axis_index("c")
        pltpu.sync_copy(cb_hbm, cb_v)
        for th, tv in zip(tabs_hbm, tabs_v):
            pltpu.sync_copy(th, tv)

        zero16 = jnp.zeros((16,), jnp.float32)

        for ci in range(CPW):
            chunk = wid * CPW + ci
            base = chunk * CHUNK
            bounds = cb_v[pl.ds(chunk, 16)]
            e0 = bounds[0]
            e1 = bounds[1]

            def zero_row(r, _):
                for j in range(D // 16):
                    acc_m[r, pl.ds(j * 16, 16)] = zero16
                if mode == "gat":
                    acc_s[r, :] = zero16
                return 0

            lax.fori_loop(0, CHUNK, zero_row, 0)

            t0 = e0 // KT
            t1 = (e1 + (KT - 1)) // KT

            def tile_body(t, _):
                eb = pl.multiple_of(t * KT, KT)
                pltpu.sync_copy(srcs_hbm.at[pl.ds(eb, KT)], idx_v)
                pltpu.sync_copy(dsts_hbm.at[pl.ds(eb, KT)],
                                dst_v.at[pl.ds(0, KT)])
                if mode == "gat":
                    for j in range(KT // 16):
                        sv = idx_v[pl.ds(j * 16, 16)]
                        dv = dst_v[pl.ds(j * 16, 16)]
                        logit = (plsc.load_gather(tabs_v[0], [sv])
                                 + plsc.load_gather(tabs_v[1], [dv]))
                        logit = jnp.where(logit > 0, logit, 0.2 * logit)
                        w_v[pl.ds(j * 16, 16)] = jnp.exp(logit)
                elif mode == "gcn":
                    for j in range(KT // 16):
                        sv = idx_v[pl.ds(j * 16, 16)]
                        w_v[pl.ds(j * 16, 16)] = plsc.load_gather(tabs_v[0], [sv])
                pltpu.async_copy(tab_hbm.at[idx_v], rows_v, sem).wait()

                lo = jnp.maximum(e0 - eb, 0)
                hi = jnp.minimum(e1 - eb, KT)

                def edge_body(k, _):
                    ld = dst_v[pl.ds(k, 16)][0] - base
                    if mode == "sage":
                        for j in range(D // 16):
                            plsc.addupdate(acc_m.at[ld, pl.ds(j * 16, 16)],
                                           rows_v[k, pl.ds(j * 16, 16)])
                    else:
                        wb = jnp.broadcast_to(w_v[pl.ds(k, 16)][0], (16,))
                        for j in range(D // 16):
                            plsc.addupdate(acc_m.at[ld, pl.ds(j * 16, 16)],
                                           wb * rows_v[k, pl.ds(j * 16, 16)])
                        if mode == "gat":
                            plsc.addupdate(acc_s.at[ld, :], wb)
                    return 0

                lax.fori_loop(lo, hi, edge_body, 0)
                return 0

            lax.fori_loop(t0, t1, tile_body, 0)

            pltpu.sync_copy(acc_m, m_hbm.at[pl.ds(base, CHUNK)])
            if mode == "gat":
                pltpu.sync_copy(acc_s, s_hbm.at[pl.ds(base, CHUNK)])

    return spmm


_SPMM_CACHE = {}


def _get_spmm(D, mode):
    key = (D, mode)
    if key not in _SPMM_CACHE:
        _SPMM_CACHE[key] = _make_spmm(D, mode)
    return _SPMM_CACHE[key]


_GRID = NP // 128


def _row_spec(width):
    if width is None:
        return pl.BlockSpec((128,), lambda i: (i,))
    return pl.BlockSpec((128, width), lambda i: (i, 0))


def _full_spec(*shape):
    if len(shape) == 1:
        return pl.BlockSpec(shape, lambda i: (0,))
    return pl.BlockSpec(shape, lambda i: (0, 0))


def _ln(v, g, b):
    m = jnp.mean(v, axis=-1, keepdims=True)
    var = jnp.mean((v - m) ** 2, axis=-1, keepdims=True)
    return (v - m) * jax.lax.rsqrt(var + 1e-5) * g + b


def _elu(v):
    return jnp.where(v > 0, v, jnp.exp(jnp.minimum(v, 0.0)) - 1.0)


def _mm(a, b):
    return jnp.dot(a, b, preferred_element_type=jnp.float32)


def _p1_body(x_ref, w1_ref, a1s_ref, a1d_ref, h_ref, asc_ref, adc_ref):
    h = _mm(x_ref[...], w1_ref[...])
    h_ref[...] = h
    asc_ref[...] = _mm(h, a1s_ref[...])
    adc_ref[...] = _mm(h, a1d_ref[...])


def _p1(xp, W1, a1s, a1d):
    return pl.pallas_call(
        _p1_body,
        grid=(_GRID,),
        in_specs=[_row_spec(F_IN), _full_spec(F_IN, H), _full_spec(H),
                  _full_spec(H)],
        out_specs=[_row_spec(H), _row_spec(None), _row_spec(None)],
        out_shape=[_f32((NP, H)), _f32((NP,)), _f32((NP,))],
    )(xp, W1, a1s, a1d)


def _gat_combine(m, s, h, asc, adc, b):
    logit = asc + adc
    wself = jnp.exp(jnp.where(logit > 0, logit, 0.2 * logit))
    num = m + wself[:, None] * h
    den = s[:, 0:1] + wself[:, None] + 1e-16
    return num / den + b


def _e1_body(m_ref, s_ref, h_ref, asc_ref, adc_ref, cnt_ref, w2_ref,
             b1_ref, g1_ref, be1_ref, h2_ref, dinv_ref):
    gat = _gat_combine(m_ref[...], s_ref[...], h_ref[...], asc_ref[...],
                       adc_ref[...], b1_ref[...])
    act = _ln(_elu(gat), g1_ref[...], be1_ref[...])
    h2_ref[...] = _mm(act, w2_ref[...])
    dinv_ref[...] = jax.lax.rsqrt(cnt_ref[...] + 1.0)


def _e1(M1, S1, h1, asc1, adc1, cntf, W2, b1, g1, be1):
    return pl.pallas_call(
        _e1_body,
        grid=(_GRID,),
        in_specs=[_row_spec(H), _row_spec(16), _row_spec(H), _row_spec(None),
                  _row_spec(None), _row_spec(None), _full_spec(H, H),
                  _full_spec(H), _full_spec(H), _full_spec(H)],
        out_specs=[_row_spec(H), _row_spec(None)],
        out_shape=[_f32((NP, H)), _f32((NP,))],
    )(M1, S1, h1, asc1, adc1, cntf, W2, b1, g1, be1)


def _e2_body(m_ref, h2_ref, dinv_ref, b2_ref, g2_ref, be2_ref, wr3_ref,
             h3_ref, r3_ref):
    dinv = dinv_ref[...][:, None]
    gcn = dinv * m_ref[...] + (dinv * dinv) * h2_ref[...] + b2_ref[...]
    h3 = _ln(_elu(gcn), g2_ref[...], be2_ref[...])
    h3_ref[...] = h3
    r3_ref[...] = _mm(h3, wr3_ref[...])


def _e2(M2, h2, dinv, b2, g2, be2, Wr3):
    return pl.pallas_call(
        _e2_body,
        grid=(_GRID,),
        in_specs=[_row_spec(H), _row_spec(H), _row_spec(None), _full_spec(H),
                  _full_spec(H), _full_spec(H), _full_spec(H, H)],
        out_specs=[_row_spec(H), _row_spec(H)],
        out_shape=[_f32((NP, H)), _f32((NP, H))],
    )(M2, h2, dinv, b2, g2, be2, Wr3)


def _e3_body(m_ref, r3_ref, cnt_ref, wl3_ref, bl3_ref, g3_ref, be3_ref,
             w4_ref, a4s_ref, a4d_ref, x_ref, wres_ref, bres_ref,
             inter_ref, h4_ref, asc_ref, adc_ref, res_ref):
    mean = m_ref[...] / jnp.maximum(cnt_ref[...], 1.0)[:, None]
    sage = _mm(mean, wl3_ref[...]) + bl3_ref[...] + r3_ref[...]
    inter = _ln(_elu(sage), g3_ref[...], be3_ref[...])
    inter_ref[...] = inter
    h4 = _mm(inter, w4_ref[...])
    h4_ref[...] = h4
    asc_ref[...] = _mm(h4, a4s_ref[...])
    adc_ref[...] = _mm(h4, a4d_ref[...])
    res_ref[...] = _mm(x_ref[...], wres_ref[...]) + bres_ref[...]


def _e3(M3, r3, cntf, Wl3, bl3, g3, be3, W4, a4s, a4d, xp, Wres, bres):
    return pl.pallas_call(
        _e3_body,
        grid=(_GRID,),
        in_specs=[_row_spec(H), _row_spec(H), _row_spec(None),
                  _full_spec(H, H), _full_spec(H), _full_spec(H),
                  _full_spec(H), _full_spec(H, C), _full_spec(C),
                  _full_spec(C), _row_spec(F_IN), _full_spec(F_IN, C),
                  _full_spec(C)],
        out_specs=[_row_spec(H), _row_spec(C), _row_spec(None),
                   _row_spec(None), _row_spec(C)],
        out_shape=[_f32((NP, H)), _f32((NP, C)), _f32((NP,)), _f32((NP,)),
                   _f32((NP, C))],
    )(M3, r3, cntf, Wl3, bl3, g3, be3, W4, a4s, a4d, xp, Wres, bres)


def _e4_body(m_ref, s_ref, h4_ref, asc_ref, adc_ref, b4_ref, res_ref,
             out_ref):
    out_ref[...] = _gat_combine(m_ref[...], s_ref[...], h4_ref[...],
                                asc_ref[...], adc_ref[...],
                                b4_ref[...]) + res_ref[...]


def _e4(M4, S4, h4, asc4, adc4, b4, res):
    return pl.pallas_call(
        _e4_body,
        grid=(_GRID,),
        in_specs=[_row_spec(C), _row_spec(16), _row_spec(C), _row_spec(None),
                  _row_spec(None), _full_spec(C), _row_spec(C)],
        out_specs=_row_spec(C),
        out_shape=_f32((NP, C)),
    )(M4, S4, h4, asc4, adc4, b4, res)


def kernel(x, edge_index, W1, a1s, a1d, b1, g1, be1, W2, b2, g2, be2,
           Wl3, bl3, Wr3, g3, be3, W4, a4s, a4d, b4, Wres, bres):
    src = edge_index[0]
    dst = edge_index[1]
    order = jnp.argsort(dst)
    src_s = jnp.take(src, order).astype(jnp.int32)
    dst_s = jnp.take(dst, order).astype(jnp.int32)
    ptr = jnp.searchsorted(
        dst_s, jnp.arange(NP + 1, dtype=jnp.int32)).astype(jnp.int32)
    cb = jnp.zeros((NCHUNK + 16,), jnp.int32)
    cb = cb.at[: NCHUNK + 1].set(ptr[:: CHUNK])
    cntf = (ptr[1:] - ptr[:-1]).astype(jnp.float32)

    xp = jnp.zeros((NP, F_IN), jnp.float32).at[:N].set(x)

    h1, asc1, adc1 = _p1(xp, W1, a1s, a1d)
    M1, S1 = _get_spmm(H, "gat")(h1, src_s, dst_s, cb, asc1, adc1)
    h2, dinv = _e1(M1, S1, h1, asc1, adc1, cntf, W2, b1, g1, be1)

    M2, = _get_spmm(H, "gcn")(h2, src_s, dst_s, cb, dinv)
    h3, r3 = _e2(M2, h2, dinv, b2, g2, be2, Wr3)

    M3, = _get_spmm(H, "sage")(h3, src_s, dst_s, cb)
    inter, h4, asc4, adc4, res = _e3(M3, r3, cntf, Wl3, bl3, g3, be3,
                                     W4, a4s, a4d, xp, Wres, bres)

    M4, S4 = _get_spmm(C, "gat")(h4, src_s, dst_s, cb, asc4, adc4)
    out = _e4(M4, S4, h4, asc4, adc4, b4, res)

    return inter[:N], out[:N]


# speedup vs baseline: 4.1460x; 1.2951x over previous
import functools

import jax
import jax.numpy as jnp
from jax import lax
from jax.experimental import pallas as pl
from jax.experimental.pallas import tpu as pltpu
from jax.experimental.pallas import tpu_sc as plsc

N = 10000
E = 160000
F_IN = 256
H = 512
C = 128

CHUNK = 80
NCHUNK = 128
NP = CHUNK * NCHUNK
KT = 32
SBMAX = 512
EP = ((E + SBMAX - 1) // SBMAX) * SBMAX
NWORK = 32
CPW = NCHUNK // NWORK


def _f32(shape):
    return jax.ShapeDtypeStruct(shape, jnp.float32)


def _make_spmm(D, mode):
    n_tab = {"gat": 2, "sage": 0}[mode]
    TPS = 16
    SB = KT * TPS
    _mesh = plsc.VectorSubcoreMesh(core_axis_name="c", subcore_axis_name="s")

    scratch = [
        pltpu.VMEM((CHUNK, D), jnp.float32),
        pltpu.VMEM((CHUNK, 16), jnp.float32),
        pltpu.VMEM((NCHUNK + 16,), jnp.int32),
        pltpu.VMEM((SB,), jnp.int32),
        pltpu.VMEM((SB + 16,), jnp.int32),
        pltpu.VMEM((SB + 16,), jnp.float32),
        pltpu.VMEM((2, KT, D), jnp.float32),
        pltpu.SemaphoreType.DMA,
        pltpu.SemaphoreType.DMA,
    ] + [pltpu.VMEM((NP,), jnp.float32)] * n_tab

    out_type = [_f32((NP, D))] + ([_f32((NP, 16))] if mode == "gat" else [])

    @functools.partial(
        pl.kernel, out_type=out_type, mesh=_mesh, scratch_types=scratch,
        compiler_params=pltpu.CompilerParams(needs_layout_passes=False))
    def spmm(tab_hbm, srcs_hbm, dsts_hbm, cb_hbm, *rest):
        tabs_hbm = rest[:n_tab]
        if mode == "gat":
            m_hbm, s_hbm = rest[n_tab:n_tab + 2]
            rest = rest[n_tab + 2:]
        else:
            m_hbm = rest[n_tab]
            rest = rest[n_tab + 1:]
        acc_m, acc_s, cb_v, idx_v, dst_v, w_v, rows_v, sem0, sem1 = rest[:9]
        tabs_v = rest[9:9 + n_tab]
        sems = (sem0, sem1)

        wid = lax.axis_index("s") * 2 + lax.axis_index("c")
        pltpu.sync_copy(cb_hbm, cb_v)
        for th, tv in zip(tabs_hbm, tabs_v):
            pltpu.sync_copy(th, tv)

        zero16 = jnp.zeros((16,), jnp.float32)

        def chunk_body(ci, _):
            chunk = wid * CPW + ci
            base = pl.multiple_of(chunk * CHUNK, CHUNK)
            bounds = cb_v[pl.ds(chunk, 16)]
            e0 = bounds[0]
            e1 = bounds[1]

            def zero_row(r, _):
                for j in range(D // 16):
                    acc_m[r, pl.ds(j * 16, 16)] = zero16
                if mode == "gat":
                    acc_s[r, :] = zero16
                return 0

            lax.fori_loop(0, CHUNK, zero_row, 0)

            s0 = e0 // SB
            s1 = (e1 + (SB - 1)) // SB

            def sb_body(sb, _):
                ebase = pl.multiple_of(sb * SB, SB)
                pltpu.sync_copy(srcs_hbm.at[pl.ds(ebase, SB)], idx_v)
                pltpu.sync_copy(dsts_hbm.at[pl.ds(ebase, SB)],
                                dst_v.at[pl.ds(0, SB)])
                if mode == "gat":
                    for j in range(SB // 16):
                        sv = idx_v[pl.ds(j * 16, 16)]
                        dv = dst_v[pl.ds(j * 16, 16)]
                        logit = (plsc.load_gather(tabs_v[0], [sv])
                                 + plsc.load_gather(tabs_v[1], [dv]))
                        logit = jnp.where(logit > 0, logit, 0.2 * logit)
                        w_v[pl.ds(j * 16, 16)] = jnp.exp(logit)

                def acc_tile(k):
                    eb = ebase + k * KT
                    lo = jnp.maximum(e0 - eb, 0)
                    hi = jnp.minimum(e1 - eb, KT)
                    buf = k % 2

                    def edge_body(jj, _):
                        kk = k * KT + jj
                        ld = dst_v[pl.ds(kk, 16)][0] - base
                        if mode == "sage":
                            for j in range(D // 16):
                                plsc.addupdate(
                                    acc_m.at[ld, pl.ds(j * 16, 16)],
                                    rows_v[buf, jj, pl.ds(j * 16, 16)])
                        else:
                            wb = jnp.broadcast_to(w_v[pl.ds(kk, 16)][0],
                                                  (16,))
                            for j in range(D // 16):
                                plsc.addupdate(
                                    acc_m.at[ld, pl.ds(j * 16, 16)],
                                    wb * rows_v[buf, jj, pl.ds(j * 16, 16)])
                            if mode == "gat":
                                plsc.addupdate(acc_s.at[ld, :], wb)
                        return 0

                    lax.fori_loop(lo, hi, edge_body, 0)

                descs = [None] * TPS
                valids = [None] * TPS
                for k in range(TPS):
                    eb = ebase + k * KT
                    valids[k] = jnp.logical_and(eb < e1, eb + KT > e0)
                    descs[k] = pltpu.make_async_copy(
                        tab_hbm.at[idx_v.at[pl.ds(k * KT, KT)]],
                        rows_v.at[k % 2], sems[k % 2])

                    @pl.when(valids[k])
                    def _(k=k):
                        descs[k].start()

                    if k > 0:
                        @pl.when(valids[k - 1])
                        def _(k=k):
                            descs[k - 1].wait()
                            acc_tile(k - 1)

                @pl.when(valids[TPS - 1])
                def _():
                    descs[TPS - 1].wait()
                    acc_tile(TPS - 1)
                return 0

            lax.fori_loop(s0, s1, sb_body, 0)

            pltpu.sync_copy(acc_m, m_hbm.at[pl.ds(base, CHUNK)])
            if mode == "gat":
                pltpu.sync_copy(acc_s, s_hbm.at[pl.ds(base, CHUNK)])
            return 0

        lax.fori_loop(0, CPW, chunk_body, 0)

    return spmm


_SPMM_CACHE = {}


def _get_spmm(D, mode):
    key = (D, mode)
    if key not in _SPMM_CACHE:
        _SPMM_CACHE[key] = _make_spmm(D, mode)
    return _SPMM_CACHE[key]


_GRID = NP // 128


def _row_spec(width):
    if width is None:
        return pl.BlockSpec((128,), lambda i: (i,))
    return pl.BlockSpec((128, width), lambda i: (i, 0))


def _full_spec(*shape):
    if len(shape) == 1:
        return pl.BlockSpec(shape, lambda i: (0,))
    return pl.BlockSpec(shape, lambda i: (0, 0))


def _ln(v, g, b):
    m = jnp.mean(v, axis=-1, keepdims=True)
    var = jnp.mean((v - m) ** 2, axis=-1, keepdims=True)
    return (v - m) * jax.lax.rsqrt(var + 1e-5) * g + b


def _elu(v):
    return jnp.where(v > 0, v, jnp.exp(jnp.minimum(v, 0.0)) - 1.0)


def _mm(a, b):
    return jnp.dot(a, b, preferred_element_type=jnp.float32)


def _p1_body(x_ref, w1_ref, a1s_ref, a1d_ref, h_ref, asc_ref, adc_ref):
    h = _mm(x_ref[...], w1_ref[...])
    h_ref[...] = h
    asc_ref[...] = _mm(h, a1s_ref[...])
    adc_ref[...] = _mm(h, a1d_ref[...])


def _p1(xp, W1, a1s, a1d):
    return pl.pallas_call(
        _p1_body,
        grid=(_GRID,),
        in_specs=[_row_spec(F_IN), _full_spec(F_IN, H), _full_spec(H),
                  _full_spec(H)],
        out_specs=[_row_spec(H), _row_spec(None), _row_spec(None)],
        out_shape=[_f32((NP, H)), _f32((NP,)), _f32((NP,))],
    )(xp, W1, a1s, a1d)


def _gat_combine(m, s, h, asc, adc, b):
    logit = asc + adc
    wself = jnp.exp(jnp.where(logit > 0, logit, 0.2 * logit))
    num = m + wself[:, None] * h
    den = s[:, 0:1] + wself[:, None] + 1e-16
    return num / den + b


def _e1_body(m_ref, s_ref, h_ref, asc_ref, adc_ref, cnt_ref, w2_ref,
             b1_ref, g1_ref, be1_ref, h2_ref, dinv_ref):
    gat = _gat_combine(m_ref[...], s_ref[...], h_ref[...], asc_ref[...],
                       adc_ref[...], b1_ref[...])
    act = _ln(_elu(gat), g1_ref[...], be1_ref[...])
    dinv = jax.lax.rsqrt(cnt_ref[...] + 1.0)
    h2_ref[...] = dinv[:, None] * _mm(act, w2_ref[...])
    dinv_ref[...] = dinv


def _e1(M1, S1, h1, asc1, adc1, cntf, W2, b1, g1, be1):
    return pl.pallas_call(
        _e1_body,
        grid=(_GRID,),
        in_specs=[_row_spec(H), _row_spec(16), _row_spec(H), _row_spec(None),
                  _row_spec(None), _row_spec(None), _full_spec(H, H),
                  _full_spec(H), _full_spec(H), _full_spec(H)],
        out_specs=[_row_spec(H), _row_spec(None)],
        out_shape=[_f32((NP, H)), _f32((NP,))],
    )(M1, S1, h1, asc1, adc1, cntf, W2, b1, g1, be1)


def _e2_body(m_ref, h2_ref, dinv_ref, b2_ref, g2_ref, be2_ref, wr3_ref,
             h3_ref, r3_ref):
    dinv = dinv_ref[...][:, None]
    gcn = dinv * (m_ref[...] + h2_ref[...]) + b2_ref[...]
    h3 = _ln(_elu(gcn), g2_ref[...], be2_ref[...])
    h3_ref[...] = h3
    r3_ref[...] = _mm(h3, wr3_ref[...])


def _e2(M2, h2, dinv, b2, g2, be2, Wr3):
    return pl.pallas_call(
        _e2_body,
        grid=(_GRID,),
        in_specs=[_row_spec(H), _row_spec(H), _row_spec(None), _full_spec(H),
                  _full_spec(H), _full_spec(H), _full_spec(H, H)],
        out_specs=[_row_spec(H), _row_spec(H)],
        out_shape=[_f32((NP, H)), _f32((NP, H))],
    )(M2, h2, dinv, b2, g2, be2, Wr3)


def _e3_body(m_ref, r3_ref, cnt_ref, wl3_ref, bl3_ref, g3_ref, be3_ref,
             w4_ref, a4s_ref, a4d_ref, x_ref, wres_ref, bres_ref,
             inter_ref, h4_ref, asc_ref, adc_ref, res_ref):
    mean = m_ref[...] / jnp.maximum(cnt_ref[...], 1.0)[:, None]
    sage = _mm(mean, wl3_ref[...]) + bl3_ref[...] + r3_ref[...]
    inter = _ln(_elu(sage), g3_ref[...], be3_ref[...])
    inter_ref[...] = inter
    h4 = _mm(inter, w4_ref[...])
    h4_ref[...] = h4
    asc_ref[...] = _mm(h4, a4s_ref[...])
    adc_ref[...] = _mm(h4, a4d_ref[...])
    res_ref[...] = _mm(x_ref[...], wres_ref[...]) + bres_ref[...]


def _e3(M3, r3, cntf, Wl3, bl3, g3, be3, W4, a4s, a4d, xp, Wres, bres):
    return pl.pallas_call(
        _e3_body,
        grid=(_GRID,),
        in_specs=[_row_spec(H), _row_spec(H), _row_spec(None),
                  _full_spec(H, H), _full_spec(H), _full_spec(H),
                  _full_spec(H), _full_spec(H, C), _full_spec(C),
                  _full_spec(C), _row_spec(F_IN), _full_spec(F_IN, C),
                  _full_spec(C)],
        out_specs=[_row_spec(H), _row_spec(C), _row_spec(None),
                   _row_spec(None), _row_spec(C)],
        out_shape=[_f32((NP, H)), _f32((NP, C)), _f32((NP,)), _f32((NP,)),
                   _f32((NP, C))],
    )(M3, r3, cntf, Wl3, bl3, g3, be3, W4, a4s, a4d, xp, Wres, bres)


def _e4_body(m_ref, s_ref, h4_ref, asc_ref, adc_ref, b4_ref, res_ref,
             out_ref):
    out_ref[...] = _gat_combine(m_ref[...], s_ref[...], h4_ref[...],
                                asc_ref[...], adc_ref[...],
                                b4_ref[...]) + res_ref[...]


def _e4(M4, S4, h4, asc4, adc4, b4, res):
    return pl.pallas_call(
        _e4_body,
        grid=(_GRID,),
        in_specs=[_row_spec(C), _row_spec(16), _row_spec(C), _row_spec(None),
                  _row_spec(None), _full_spec(C), _row_spec(C)],
        out_specs=_row_spec(C),
        out_shape=_f32((NP, C)),
    )(M4, S4, h4, asc4, adc4, b4, res)


def kernel(x, edge_index, W1, a1s, a1d, b1, g1, be1, W2, b2, g2, be2,
           Wl3, bl3, Wr3, g3, be3, W4, a4s, a4d, b4, Wres, bres):
    src = edge_index[0]
    dst = edge_index[1]
    order = jnp.argsort(dst)
    src_s = jnp.take(src, order).astype(jnp.int32)
    dst_s = jnp.take(dst, order).astype(jnp.int32)
    ptr = jnp.searchsorted(
        dst_s, jnp.arange(NP + 1, dtype=jnp.int32)).astype(jnp.int32)
    src_s = jnp.zeros((EP,), jnp.int32).at[:E].set(src_s)
    dst_s = jnp.zeros((EP,), jnp.int32).at[:E].set(dst_s)
    cb = jnp.zeros((NCHUNK + 16,), jnp.int32)
    cb = cb.at[: NCHUNK + 1].set(ptr[:: CHUNK])
    cntf = (ptr[1:] - ptr[:-1]).astype(jnp.float32)

    xp = jnp.zeros((NP, F_IN), jnp.float32).at[:N].set(x)

    h1, asc1, adc1 = _p1(xp, W1, a1s, a1d)
    M1, S1 = _get_spmm(H, "gat")(h1, src_s, dst_s, cb, asc1, adc1)
    h2, dinv = _e1(M1, S1, h1, asc1, adc1, cntf, W2, b1, g1, be1)

    M2, = _get_spmm(H, "sage")(h2, src_s, dst_s, cb)
    h3, r3 = _e2(M2, h2, dinv, b2, g2, be2, Wr3)

    M3, = _get_spmm(H, "sage")(h3, src_s, dst_s, cb)
    inter, h4, asc4, adc4, res = _e3(M3, r3, cntf, Wl3, bl3, g3, be3,
                                     W4, a4s, a4d, xp, Wres, bres)

    M4, S4 = _get_spmm(C, "gat")(h4, src_s, dst_s, cb, asc4, adc4)
    out = _e4(M4, S4, h4, asc4, adc4, b4, res)

    return inter[:N], out[:N]


# speedup vs baseline: 4.1568x; 1.0026x over previous
import functools

import jax
import jax.numpy as jnp
from jax import lax
from jax.experimental import pallas as pl
from jax.experimental.pallas import tpu as pltpu
from jax.experimental.pallas import tpu_sc as plsc

N = 10000
E = 160000
F_IN = 256
H = 512
C = 128

CHUNK = 80
NCHUNK = 128
NP = CHUNK * NCHUNK
KT = 32
SBMAX = 512
EP = ((E + SBMAX - 1) // SBMAX) * SBMAX
NWORK = 32
CPW = NCHUNK // NWORK


def _f32(shape):
    return jax.ShapeDtypeStruct(shape, jnp.float32)


def _make_spmm(D, mode):
    n_tab = {"gat": 2, "sage": 0}[mode]
    TPS = 16
    SB = KT * TPS
    _mesh = plsc.VectorSubcoreMesh(core_axis_name="c", subcore_axis_name="s")

    scratch = [
        pltpu.VMEM((CHUNK, D), jnp.float32),
        pltpu.VMEM((CHUNK, 16), jnp.float32),
        pltpu.VMEM((NCHUNK + 16,), jnp.int32),
        pltpu.VMEM((SB,), jnp.int32),
        pltpu.VMEM((SB + 16,), jnp.int32),
        pltpu.VMEM((SB + 16,), jnp.float32),
        pltpu.VMEM((2, KT, D), jnp.float32),
        pltpu.SemaphoreType.DMA,
        pltpu.SemaphoreType.DMA,
    ] + [pltpu.VMEM((NP,), jnp.float32)] * n_tab

    out_type = [_f32((NP, D))] + ([_f32((NP, 16))] if mode == "gat" else [])

    @functools.partial(
        pl.kernel, out_type=out_type, mesh=_mesh, scratch_types=scratch,
        compiler_params=pltpu.CompilerParams(needs_layout_passes=False))
    def spmm(tab_hbm, srcs_hbm, dsts_hbm, cb_hbm, *rest):
        tabs_hbm = rest[:n_tab]
        if mode == "gat":
            m_hbm, s_hbm = rest[n_tab:n_tab + 2]
            rest = rest[n_tab + 2:]
        else:
            m_hbm = rest[n_tab]
            rest = rest[n_tab + 1:]
        acc_m, acc_s, cb_v, idx_v, dst_v, w_v, rows_v, sem0, sem1 = rest[:9]
        tabs_v = rest[9:9 + n_tab]
        sems = (sem0, sem1)

        wid = lax.axis_index("s") * 2 + lax.axis_index("c")
        pltpu.sync_copy(cb_hbm, cb_v)
        for th, tv in zip(tabs_hbm, tabs_v):
            pltpu.sync_copy(th, tv)

        zero16 = jnp.zeros((16,), jnp.float32)

        def chunk_body(ci, _):
            chunk = wid * CPW + ci
            base = pl.multiple_of(chunk * CHUNK, CHUNK)
            bounds = cb_v[pl.ds(chunk, 16)]
            e0 = bounds[0]
            e1 = bounds[1]

            def zero_row(r, _):
                for j in range(D // 16):
                    acc_m[r, pl.ds(j * 16, 16)] = zero16
                if mode == "gat":
                    acc_s[r, :] = zero16
                return 0

            lax.fori_loop(0, CHUNK, zero_row, 0)

            s0 = e0 // SB
            s1 = (e1 + (SB - 1)) // SB

            def sb_body(sb, _):
                ebase = pl.multiple_of(sb * SB, SB)
                pltpu.sync_copy(srcs_hbm.at[pl.ds(ebase, SB)], idx_v)
                pltpu.sync_copy(dsts_hbm.at[pl.ds(ebase, SB)],
                                dst_v.at[pl.ds(0, SB)])
                if mode == "gat":
                    for j in range(SB // 16):
                        sv = idx_v[pl.ds(j * 16, 16)]
                        dv = dst_v[pl.ds(j * 16, 16)]
                        logit = (plsc.load_gather(tabs_v[0], [sv])
                                 + plsc.load_gather(tabs_v[1], [dv]))
                        logit = jnp.where(logit > 0, logit, 0.2 * logit)
                        w_v[pl.ds(j * 16, 16)] = jnp.exp(logit)

                def acc_tile(k):
                    eb = ebase + k * KT
                    lo = jnp.maximum(e0 - eb, 0)
                    hi = jnp.minimum(e1 - eb, KT)
                    buf = k % 2

                    def edge_body(jj, _):
                        kk = k * KT + jj
                        ld = dst_v[pl.ds(kk, 16)][0] - base
                        if mode == "sage":
                            for j in range(D // 16):
                                plsc.addupdate(
                                    acc_m.at[ld, pl.ds(j * 16, 16)],
                                    rows_v[buf, jj, pl.ds(j * 16, 16)])
                        else:
                            wb = jnp.broadcast_to(w_v[pl.ds(kk, 16)][0],
                                                  (16,))
                            for j in range(D // 16):
                                plsc.addupdate(
                                    acc_m.at[ld, pl.ds(j * 16, 16)],
                                    wb * rows_v[buf, jj, pl.ds(j * 16, 16)])
                            if mode == "gat":
                                plsc.addupdate(acc_s.at[ld, :], wb)
                        return 0

                    lax.fori_loop(lo, hi, edge_body, 0)

                descs = [None] * TPS
                valids = [None] * TPS
                for k in range(TPS):
                    eb = ebase + k * KT
                    valids[k] = jnp.logical_and(eb < e1, eb + KT > e0)
                    descs[k] = pltpu.make_async_copy(
                        tab_hbm.at[idx_v.at[pl.ds(k * KT, KT)]],
                        rows_v.at[k % 2], sems[k % 2])

                    @pl.when(valids[k])
                    def _(k=k):
                        descs[k].start()

                    if k > 0:
                        @pl.when(valids[k - 1])
                        def _(k=k):
                            descs[k - 1].wait()
                            acc_tile(k - 1)

                @pl.when(valids[TPS - 1])
                def _():
                    descs[TPS - 1].wait()
                    acc_tile(TPS - 1)
                return 0

            lax.fori_loop(s0, s1, sb_body, 0)

            pltpu.sync_copy(acc_m, m_hbm.at[pl.ds(base, CHUNK)])
            if mode == "gat":
                pltpu.sync_copy(acc_s, s_hbm.at[pl.ds(base, CHUNK)])
            return 0

        lax.fori_loop(0, CPW, chunk_body, 0)

    return spmm


_SPMM_CACHE = {}


def _get_spmm(D, mode):
    key = (D, mode)
    if key not in _SPMM_CACHE:
        _SPMM_CACHE[key] = _make_spmm(D, mode)
    return _SPMM_CACHE[key]


_GRID = NP // 128


def _row_spec(width):
    if width is None:
        return pl.BlockSpec((128,), lambda i: (i,))
    return pl.BlockSpec((128, width), lambda i: (i, 0))


def _full_spec(*shape):
    if len(shape) == 1:
        return pl.BlockSpec(shape, lambda i: (0,))
    return pl.BlockSpec(shape, lambda i: (0, 0))


def _ln(v, g, b):
    m = jnp.mean(v, axis=-1, keepdims=True)
    var = jnp.mean((v - m) ** 2, axis=-1, keepdims=True)
    return (v - m) * jax.lax.rsqrt(var + 1e-5) * g + b


def _elu(v):
    return jnp.where(v > 0, v, jnp.exp(jnp.minimum(v, 0.0)) - 1.0)


def _mm(a, b):
    return jnp.dot(a, b, preferred_element_type=jnp.float32)


def _p1_body(x_ref, w1_ref, a1s_ref, a1d_ref, h_ref, asc_ref, adc_ref):
    h = _mm(x_ref[...], w1_ref[...])
    h_ref[...] = h
    asc_ref[...] = _mm(h, a1s_ref[...])
    adc_ref[...] = _mm(h, a1d_ref[...])


def _p1(xp, W1, a1s, a1d):
    return pl.pallas_call(
        _p1_body,
        grid=(_GRID,),
        in_specs=[_row_spec(F_IN), _full_spec(F_IN, H), _full_spec(H),
                  _full_spec(H)],
        out_specs=[_row_spec(H), _row_spec(None), _row_spec(None)],
        out_shape=[_f32((NP, H)), _f32((NP,)), _f32((NP,))],
    )(xp, W1, a1s, a1d)


def _gat_combine(m, s, h, asc, adc, b):
    logit = asc + adc
    wself = jnp.exp(jnp.where(logit > 0, logit, 0.2 * logit))
    num = m + wself[:, None] * h
    den = s[:, 0:1] + wself[:, None] + 1e-16
    return num / den + b


def _e1_body(m_ref, s_ref, h_ref, asc_ref, adc_ref, cnt_ref, w2_ref,
             b1_ref, g1_ref, be1_ref, h2_ref, dinv_ref):
    gat = _gat_combine(m_ref[...], s_ref[...], h_ref[...], asc_ref[...],
                       adc_ref[...], b1_ref[...])
    act = _ln(_elu(gat), g1_ref[...], be1_ref[...])
    dinv = jax.lax.rsqrt(cnt_ref[...] + 1.0)
    h2_ref[...] = dinv[:, None] * _mm(act, w2_ref[...])
    dinv_ref[...] = dinv


def _e1(M1, S1, h1, asc1, adc1, cntf, W2, b1, g1, be1):
    return pl.pallas_call(
        _e1_body,
        grid=(_GRID,),
        in_specs=[_row_spec(H), _row_spec(16), _row_spec(H), _row_spec(None),
                  _row_spec(None), _row_spec(None), _full_spec(H, H),
                  _full_spec(H), _full_spec(H), _full_spec(H)],
        out_specs=[_row_spec(H), _row_spec(None)],
        out_shape=[_f32((NP, H)), _f32((NP,))],
    )(M1, S1, h1, asc1, adc1, cntf, W2, b1, g1, be1)


def _e2_body(m_ref, h2_ref, dinv_ref, b2_ref, g2_ref, be2_ref, wr3_ref,
             h3_ref, r3_ref):
    dinv = dinv_ref[...][:, None]
    gcn = dinv * (m_ref[...] + h2_ref[...]) + b2_ref[...]
    h3 = _ln(_elu(gcn), g2_ref[...], be2_ref[...])
    h3_ref[...] = h3
    r3_ref[...] = _mm(h3, wr3_ref[...])


def _e2(M2, h2, dinv, b2, g2, be2, Wr3):
    return pl.pallas_call(
        _e2_body,
        grid=(_GRID,),
        in_specs=[_row_spec(H), _row_spec(H), _row_spec(None), _full_spec(H),
                  _full_spec(H), _full_spec(H), _full_spec(H, H)],
        out_specs=[_row_spec(H), _row_spec(H)],
        out_shape=[_f32((NP, H)), _f32((NP, H))],
    )(M2, h2, dinv, b2, g2, be2, Wr3)


def _e3_body(m_ref, r3_ref, cnt_ref, wl3_ref, bl3_ref, g3_ref, be3_ref,
             w4_ref, a4s_ref, a4d_ref, x_ref, wres_ref, bres_ref,
             inter_ref, h4_ref, asc_ref, adc_ref, res_ref):
    mean = m_ref[...] / jnp.maximum(cnt_ref[...], 1.0)[:, None]
    sage = _mm(mean, wl3_ref[...]) + bl3_ref[...] + r3_ref[...]
    inter = _ln(_elu(sage), g3_ref[...], be3_ref[...])
    inter_ref[...] = inter
    h4 = _mm(inter, w4_ref[...])
    h4_ref[...] = h4
    asc_ref[...] = _mm(h4, a4s_ref[...])
    adc_ref[...] = _mm(h4, a4d_ref[...])
    res_ref[...] = _mm(x_ref[...], wres_ref[...]) + bres_ref[...]


def _e3(M3, r3, cntf, Wl3, bl3, g3, be3, W4, a4s, a4d, xp, Wres, bres):
    return pl.pallas_call(
        _e3_body,
        grid=(_GRID,),
        in_specs=[_row_spec(H), _row_spec(H), _row_spec(None),
                  _full_spec(H, H), _full_spec(H), _full_spec(H),
                  _full_spec(H), _full_spec(H, C), _full_spec(C),
                  _full_spec(C), _row_spec(F_IN), _full_spec(F_IN, C),
                  _full_spec(C)],
        out_specs=[_row_spec(H), _row_spec(C), _row_spec(None),
                   _row_spec(None), _row_spec(C)],
        out_shape=[_f32((NP, H)), _f32((NP, C)), _f32((NP,)), _f32((NP,)),
                   _f32((NP, C))],
    )(M3, r3, cntf, Wl3, bl3, g3, be3, W4, a4s, a4d, xp, Wres, bres)


def _e4_body(m_ref, s_ref, h4_ref, asc_ref, adc_ref, b4_ref, res_ref,
             out_ref):
    out_ref[...] = _gat_combine(m_ref[...], s_ref[...], h4_ref[...],
                                asc_ref[...], adc_ref[...],
                                b4_ref[...]) + res_ref[...]


def _e4(M4, S4, h4, asc4, adc4, b4, res):
    return pl.pallas_call(
        _e4_body,
        grid=(_GRID,),
        in_specs=[_row_spec(C), _row_spec(16), _row_spec(C), _row_spec(None),
                  _row_spec(None), _full_spec(C), _row_spec(C)],
        out_specs=_row_spec(C),
        out_shape=_f32((NP, C)),
    )(M4, S4, h4, asc4, adc4, b4, res)


def kernel(x, edge_index, W1, a1s, a1d, b1, g1, be1, W2, b2, g2, be2,
           Wl3, bl3, Wr3, g3, be3, W4, a4s, a4d, b4, Wres, bres):
    src = edge_index[0]
    dst = edge_index[1]
    eidx = jnp.arange(E, dtype=jnp.uint32)
    key = jnp.sort(dst.astype(jnp.uint32) * jnp.uint32(1 << 18) + eidx)
    dst_s = (key >> 18).astype(jnp.int32)
    order = (key & jnp.uint32((1 << 18) - 1)).astype(jnp.int32)
    src_s = jnp.take(src, order).astype(jnp.int32)
    ptr = jnp.searchsorted(
        dst_s, jnp.arange(NP + 1, dtype=jnp.int32)).astype(jnp.int32)
    src_s = jnp.zeros((EP,), jnp.int32).at[:E].set(src_s)
    dst_s = jnp.zeros((EP,), jnp.int32).at[:E].set(dst_s)
    cb = jnp.zeros((NCHUNK + 16,), jnp.int32)
    cb = cb.at[: NCHUNK + 1].set(ptr[:: CHUNK])
    cntf = (ptr[1:] - ptr[:-1]).astype(jnp.float32)

    xp = jnp.zeros((NP, F_IN), jnp.float32).at[:N].set(x)

    h1, asc1, adc1 = _p1(xp, W1, a1s, a1d)
    M1, S1 = _get_spmm(H, "gat")(h1, src_s, dst_s, cb, asc1, adc1)
    h2, dinv = _e1(M1, S1, h1, asc1, adc1, cntf, W2, b1, g1, be1)

    M2, = _get_spmm(H, "sage")(h2, src_s, dst_s, cb)
    h3, r3 = _e2(M2, h2, dinv, b2, g2, be2, Wr3)

    M3, = _get_spmm(H, "sage")(h3, src_s, dst_s, cb)
    inter, h4, asc4, adc4, res = _e3(M3, r3, cntf, Wl3, bl3, g3, be3,
                                     W4, a4s, a4d, xp, Wres, bres)

    M4, S4 = _get_spmm(C, "gat")(h4, src_s, dst_s, cb, asc4, adc4)
    out = _e4(M4, S4, h4, asc4, adc4, b4, res)

    return inter[:N], out[:N]


# speedup vs baseline: 4.3554x; 1.0478x over previous
import functools

import jax
import jax.numpy as jnp
from jax import lax
from jax.experimental import pallas as pl
from jax.experimental.pallas import tpu as pltpu
from jax.experimental.pallas import tpu_sc as plsc

N = 10000
E = 160000
F_IN = 256
H = 512
C = 128

CHUNK = 80
NCHUNK = 128
NP = CHUNK * NCHUNK
KT = 32
SBMAX = 512
EP = ((E + SBMAX - 1) // SBMAX) * SBMAX
NWORK = 32
CPW = NCHUNK // NWORK


def _f32(shape):
    return jax.ShapeDtypeStruct(shape, jnp.float32)


def _make_spmm(D, mode):
    n_tab = {"gat": 2, "sage": 0}[mode]
    TPS = 16
    SB = KT * TPS
    _mesh = plsc.VectorSubcoreMesh(core_axis_name="c", subcore_axis_name="s")

    scratch = [
        pltpu.VMEM((CHUNK, D), jnp.float32),
        pltpu.VMEM((CHUNK, 16), jnp.float32),
        pltpu.VMEM((NCHUNK + 16,), jnp.int32),
        pltpu.VMEM((SB,), jnp.int32),
        pltpu.VMEM((SB + 16,), jnp.int32),
        pltpu.VMEM((SB + 16,), jnp.float32),
        pltpu.VMEM((2, KT, D), jnp.float32),
        pltpu.SemaphoreType.DMA,
        pltpu.SemaphoreType.DMA,
    ] + [pltpu.VMEM((NP,), jnp.float32)] * n_tab

    out_type = [_f32((NP, D))] + ([_f32((NP, 16))] if mode == "gat" else [])

    @functools.partial(
        pl.kernel, out_type=out_type, mesh=_mesh, scratch_types=scratch,
        compiler_params=pltpu.CompilerParams(needs_layout_passes=False))
    def spmm(tab_hbm, srcs_hbm, dsts_hbm, cb_hbm, *rest):
        tabs_hbm = rest[:n_tab]
        if mode == "gat":
            m_hbm, s_hbm = rest[n_tab:n_tab + 2]
            rest = rest[n_tab + 2:]
        else:
            m_hbm = rest[n_tab]
            rest = rest[n_tab + 1:]
        acc_m, acc_s, cb_v, idx_v, dst_v, w_v, rows_v, sem0, sem1 = rest[:9]
        tabs_v = rest[9:9 + n_tab]
        sems = (sem0, sem1)

        wid = lax.axis_index("s") * 2 + lax.axis_index("c")
        pltpu.sync_copy(cb_hbm, cb_v)
        for th, tv in zip(tabs_hbm, tabs_v):
            pltpu.sync_copy(th, tv)

        zero16 = jnp.zeros((16,), jnp.float32)

        def chunk_body(ci, _):
            chunk = wid * CPW + ci
            base = pl.multiple_of(chunk * CHUNK, CHUNK)
            bounds = cb_v[pl.ds(chunk, 16)]
            e0 = bounds[0]
            e1 = bounds[1]

            def zero_row(r, _):
                for j in range(D // 16):
                    acc_m[r, pl.ds(j * 16, 16)] = zero16
                if mode == "gat":
                    acc_s[r, :] = zero16
                return 0

            lax.fori_loop(0, CHUNK, zero_row, 0)

            s0 = e0 // SB
            s1 = (e1 + (SB - 1)) // SB

            def sb_body(sb, _):
                ebase = pl.multiple_of(sb * SB, SB)
                pltpu.sync_copy(srcs_hbm.at[pl.ds(ebase, SB)], idx_v)
                pltpu.sync_copy(dsts_hbm.at[pl.ds(ebase, SB)],
                                dst_v.at[pl.ds(0, SB)])
                if mode == "gat":
                    for j in range(SB // 16):
                        sv = idx_v[pl.ds(j * 16, 16)]
                        dv = dst_v[pl.ds(j * 16, 16)]
                        logit = (plsc.load_gather(tabs_v[0], [sv])
                                 + plsc.load_gather(tabs_v[1], [dv]))
                        logit = jnp.where(logit > 0, logit, 0.2 * logit)
                        w_v[pl.ds(j * 16, 16)] = jnp.exp(logit)

                def acc_tile(k):
                    eb = ebase + k * KT
                    lo = jnp.maximum(e0 - eb, 0)
                    hi = jnp.minimum(e1 - eb, KT)
                    buf = k % 2

                    def edge_body(jj, _):
                        kk = k * KT + jj
                        ld = dst_v[pl.ds(kk, 16)][0] - base
                        if mode == "sage":
                            for j in range(D // 16):
                                plsc.addupdate(
                                    acc_m.at[ld, pl.ds(j * 16, 16)],
                                    rows_v[buf, jj, pl.ds(j * 16, 16)])
                        else:
                            wb = jnp.broadcast_to(w_v[pl.ds(kk, 16)][0],
                                                  (16,))
                            for j in range(D // 16):
                                plsc.addupdate(
                                    acc_m.at[ld, pl.ds(j * 16, 16)],
                                    wb * rows_v[buf, jj, pl.ds(j * 16, 16)])
                            if mode == "gat":
                                plsc.addupdate(acc_s.at[ld, :], wb)
                        return 0

                    lax.fori_loop(lo, hi, edge_body, 0)

                descs = [None] * TPS
                valids = [None] * TPS
                for k in range(TPS):
                    eb = ebase + k * KT
                    valids[k] = jnp.logical_and(eb < e1, eb + KT > e0)
                    descs[k] = pltpu.make_async_copy(
                        tab_hbm.at[idx_v.at[pl.ds(k * KT, KT)]],
                        rows_v.at[k % 2], sems[k % 2])

                    @pl.when(valids[k])
                    def _(k=k):
                        descs[k].start()

                    if k > 0:
                        @pl.when(valids[k - 1])
                        def _(k=k):
                            descs[k - 1].wait()
                            acc_tile(k - 1)

                @pl.when(valids[TPS - 1])
                def _():
                    descs[TPS - 1].wait()
                    acc_tile(TPS - 1)
                return 0

            lax.fori_loop(s0, s1, sb_body, 0)

            pltpu.sync_copy(acc_m, m_hbm.at[pl.ds(base, CHUNK)])
            if mode == "gat":
                pltpu.sync_copy(acc_s, s_hbm.at[pl.ds(base, CHUNK)])
            return 0

        lax.fori_loop(0, CPW, chunk_body, 0)

    return spmm


_SPMM_CACHE = {}


def _get_spmm(D, mode):
    key = (D, mode)
    if key not in _SPMM_CACHE:
        _SPMM_CACHE[key] = _make_spmm(D, mode)
    return _SPMM_CACHE[key]


_RBLK = 512
_GRID = NP // _RBLK


def _row_spec(width):
    if width is None:
        return pl.BlockSpec((_RBLK,), lambda i: (i,))
    return pl.BlockSpec((_RBLK, width), lambda i: (i, 0))


def _full_spec(*shape):
    if len(shape) == 1:
        return pl.BlockSpec(shape, lambda i: (0,))
    return pl.BlockSpec(shape, lambda i: (0, 0))


def _ln(v, g, b):
    m = jnp.mean(v, axis=-1, keepdims=True)
    var = jnp.mean((v - m) ** 2, axis=-1, keepdims=True)
    return (v - m) * jax.lax.rsqrt(var + 1e-5) * g + b


def _elu(v):
    return jnp.where(v > 0, v, jnp.exp(jnp.minimum(v, 0.0)) - 1.0)


def _mm(a, b):
    return jnp.dot(a, b, preferred_element_type=jnp.float32)


def _p1_body(x_ref, w1_ref, a1s_ref, a1d_ref, h_ref, asc_ref, adc_ref):
    h = _mm(x_ref[...], w1_ref[...])
    h_ref[...] = h
    asc_ref[...] = _mm(h, a1s_ref[...])
    adc_ref[...] = _mm(h, a1d_ref[...])


def _p1(xp, W1, a1s, a1d):
    return pl.pallas_call(
        _p1_body,
        grid=(_GRID,),
        in_specs=[_row_spec(F_IN), _full_spec(F_IN, H), _full_spec(H),
                  _full_spec(H)],
        out_specs=[_row_spec(H), _row_spec(None), _row_spec(None)],
        out_shape=[_f32((NP, H)), _f32((NP,)), _f32((NP,))],
    )(xp, W1, a1s, a1d)


def _gat_combine(m, s, h, asc, adc, b):
    logit = asc + adc
    wself = jnp.exp(jnp.where(logit > 0, logit, 0.2 * logit))
    num = m + wself[:, None] * h
    den = s[:, 0:1] + wself[:, None] + 1e-16
    return num / den + b


def _e1_body(m_ref, s_ref, h_ref, asc_ref, adc_ref, cnt_ref, w2_ref,
             b1_ref, g1_ref, be1_ref, h2_ref, dinv_ref):
    gat = _gat_combine(m_ref[...], s_ref[...], h_ref[...], asc_ref[...],
                       adc_ref[...], b1_ref[...])
    act = _ln(_elu(gat), g1_ref[...], be1_ref[...])
    dinv = jax.lax.rsqrt(cnt_ref[...] + 1.0)
    h2_ref[...] = dinv[:, None] * _mm(act, w2_ref[...])
    dinv_ref[...] = dinv


def _e1(M1, S1, h1, asc1, adc1, cntf, W2, b1, g1, be1):
    return pl.pallas_call(
        _e1_body,
        grid=(_GRID,),
        in_specs=[_row_spec(H), _row_spec(16), _row_spec(H), _row_spec(None),
                  _row_spec(None), _row_spec(None), _full_spec(H, H),
                  _full_spec(H), _full_spec(H), _full_spec(H)],
        out_specs=[_row_spec(H), _row_spec(None)],
        out_shape=[_f32((NP, H)), _f32((NP,))],
    )(M1, S1, h1, asc1, adc1, cntf, W2, b1, g1, be1)


def _e2_body(m_ref, h2_ref, dinv_ref, b2_ref, g2_ref, be2_ref, wr3_ref,
             h3_ref, r3_ref):
    dinv = dinv_ref[...][:, None]
    gcn = dinv * (m_ref[...] + h2_ref[...]) + b2_ref[...]
    h3 = _ln(_elu(gcn), g2_ref[...], be2_ref[...])
    h3_ref[...] = h3
    r3_ref[...] = _mm(h3, wr3_ref[...])


def _e2(M2, h2, dinv, b2, g2, be2, Wr3):
    return pl.pallas_call(
        _e2_body,
        grid=(_GRID,),
        in_specs=[_row_spec(H), _row_spec(H), _row_spec(None), _full_spec(H),
                  _full_spec(H), _full_spec(H), _full_spec(H, H)],
        out_specs=[_row_spec(H), _row_spec(H)],
        out_shape=[_f32((NP, H)), _f32((NP, H))],
    )(M2, h2, dinv, b2, g2, be2, Wr3)


def _e3_body(m_ref, r3_ref, cnt_ref, wl3_ref, bl3_ref, g3_ref, be3_ref,
             w4_ref, a4s_ref, a4d_ref, x_ref, wres_ref, bres_ref,
             inter_ref, h4_ref, asc_ref, adc_ref, res_ref):
    mean = m_ref[...] / jnp.maximum(cnt_ref[...], 1.0)[:, None]
    sage = _mm(mean, wl3_ref[...]) + bl3_ref[...] + r3_ref[...]
    inter = _ln(_elu(sage), g3_ref[...], be3_ref[...])
    inter_ref[...] = inter
    h4 = _mm(inter, w4_ref[...])
    h4_ref[...] = h4
    asc_ref[...] = _mm(h4, a4s_ref[...])
    adc_ref[...] = _mm(h4, a4d_ref[...])
    res_ref[...] = _mm(x_ref[...], wres_ref[...]) + bres_ref[...]


def _e3(M3, r3, cntf, Wl3, bl3, g3, be3, W4, a4s, a4d, xp, Wres, bres):
    return pl.pallas_call(
        _e3_body,
        grid=(_GRID,),
        in_specs=[_row_spec(H), _row_spec(H), _row_spec(None),
                  _full_spec(H, H), _full_spec(H), _full_spec(H),
                  _full_spec(H), _full_spec(H, C), _full_spec(C),
                  _full_spec(C), _row_spec(F_IN), _full_spec(F_IN, C),
                  _full_spec(C)],
        out_specs=[_row_spec(H), _row_spec(C), _row_spec(None),
                   _row_spec(None), _row_spec(C)],
        out_shape=[_f32((NP, H)), _f32((NP, C)), _f32((NP,)), _f32((NP,)),
                   _f32((NP, C))],
    )(M3, r3, cntf, Wl3, bl3, g3, be3, W4, a4s, a4d, xp, Wres, bres)


def _e4_body(m_ref, s_ref, h4_ref, asc_ref, adc_ref, b4_ref, res_ref,
             out_ref):
    out_ref[...] = _gat_combine(m_ref[...], s_ref[...], h4_ref[...],
                                asc_ref[...], adc_ref[...],
                                b4_ref[...]) + res_ref[...]


def _e4(M4, S4, h4, asc4, adc4, b4, res):
    return pl.pallas_call(
        _e4_body,
        grid=(_GRID,),
        in_specs=[_row_spec(C), _row_spec(16), _row_spec(C), _row_spec(None),
                  _row_spec(None), _full_spec(C), _row_spec(C)],
        out_specs=_row_spec(C),
        out_shape=_f32((NP, C)),
    )(M4, S4, h4, asc4, adc4, b4, res)


def kernel(x, edge_index, W1, a1s, a1d, b1, g1, be1, W2, b2, g2, be2,
           Wl3, bl3, Wr3, g3, be3, W4, a4s, a4d, b4, Wres, bres):
    src = edge_index[0]
    dst = edge_index[1]
    eidx = jnp.arange(E, dtype=jnp.uint32)
    key = jnp.sort(dst.astype(jnp.uint32) * jnp.uint32(1 << 18) + eidx)
    dst_s = (key >> 18).astype(jnp.int32)
    order = (key & jnp.uint32((1 << 18) - 1)).astype(jnp.int32)
    src_s = jnp.take(src, order).astype(jnp.int32)
    ptr = jnp.searchsorted(
        dst_s, jnp.arange(NP + 1, dtype=jnp.int32)).astype(jnp.int32)
    src_s = jnp.zeros((EP,), jnp.int32).at[:E].set(src_s)
    dst_s = jnp.zeros((EP,), jnp.int32).at[:E].set(dst_s)
    cb = jnp.zeros((NCHUNK + 16,), jnp.int32)
    cb = cb.at[: NCHUNK + 1].set(ptr[:: CHUNK])
    cntf = (ptr[1:] - ptr[:-1]).astype(jnp.float32)

    xp = jnp.zeros((NP, F_IN), jnp.float32).at[:N].set(x)

    h1, asc1, adc1 = _p1(xp, W1, a1s, a1d)
    M1, S1 = _get_spmm(H, "gat")(h1, src_s, dst_s, cb, asc1, adc1)
    h2, dinv = _e1(M1, S1, h1, asc1, adc1, cntf, W2, b1, g1, be1)

    M2, = _get_spmm(H, "sage")(h2, src_s, dst_s, cb)
    h3, r3 = _e2(M2, h2, dinv, b2, g2, be2, Wr3)

    M3, = _get_spmm(H, "sage")(h3, src_s, dst_s, cb)
    inter, h4, asc4, adc4, res = _e3(M3, r3, cntf, Wl3, bl3, g3, be3,
                                     W4, a4s, a4d, xp, Wres, bres)

    M4, S4 = _get_spmm(C, "gat")(h4, src_s, dst_s, cb, asc4, adc4)
    out = _e4(M4, S4, h4, asc4, adc4, b4, res)

    return inter[:N], out[:N]


# speedup vs baseline: 4.3591x; 1.0009x over previous
import functools

import jax
import jax.numpy as jnp
from jax import lax
from jax.experimental import pallas as pl
from jax.experimental.pallas import tpu as pltpu
from jax.experimental.pallas import tpu_sc as plsc

N = 10000
E = 160000
F_IN = 256
H = 512
C = 128

CHUNK = 80
NCHUNK = 128
NP = CHUNK * NCHUNK
KT = 32
SBMAX = 512
EP = ((E + SBMAX - 1) // SBMAX) * SBMAX
NWORK = 32
CPW = NCHUNK // NWORK


def _f32(shape):
    return jax.ShapeDtypeStruct(shape, jnp.float32)


def _make_spmm(D, mode):
    n_tab = {"gat": 2, "sage": 0}[mode]
    TPS = 16
    SB = KT * TPS
    _mesh = plsc.VectorSubcoreMesh(core_axis_name="c", subcore_axis_name="s")

    scratch = [
        pltpu.VMEM((CHUNK, D), jnp.float32),
        pltpu.VMEM((CHUNK, 16), jnp.float32),
        pltpu.VMEM((NCHUNK + 16,), jnp.int32),
        pltpu.VMEM((SB,), jnp.int32),
        pltpu.VMEM((SB + 16,), jnp.int32),
        pltpu.VMEM((SB + 16,), jnp.float32),
        pltpu.VMEM((2, KT, D), jnp.float32),
        pltpu.SemaphoreType.DMA,
        pltpu.SemaphoreType.DMA,
    ] + [pltpu.VMEM((NP,), jnp.float32)] * n_tab

    out_type = [_f32((NP, D))] + ([_f32((NP, 16))] if mode == "gat" else [])

    @functools.partial(
        pl.kernel, out_type=out_type, mesh=_mesh, scratch_types=scratch,
        compiler_params=pltpu.CompilerParams(needs_layout_passes=False))
    def spmm(tab_hbm, srcs_hbm, dsts_hbm, cb_hbm, *rest):
        tabs_hbm = rest[:n_tab]
        if mode == "gat":
            m_hbm, s_hbm = rest[n_tab:n_tab + 2]
            rest = rest[n_tab + 2:]
        else:
            m_hbm = rest[n_tab]
            rest = rest[n_tab + 1:]
        acc_m, acc_s, cb_v, idx_v, dst_v, w_v, rows_v, sem0, sem1 = rest[:9]
        tabs_v = rest[9:9 + n_tab]
        sems = (sem0, sem1)

        wid = lax.axis_index("s") * 2 + lax.axis_index("c")
        pltpu.sync_copy(cb_hbm, cb_v)
        for th, tv in zip(tabs_hbm, tabs_v):
            pltpu.sync_copy(th, tv)

        zero16 = jnp.zeros((16,), jnp.float32)

        def chunk_body(ci, _):
            chunk = wid * CPW + ci
            base = pl.multiple_of(chunk * CHUNK, CHUNK)
            bounds = cb_v[pl.ds(chunk, 16)]
            e0 = bounds[0]
            e1 = bounds[1]

            def zero_row(r, _):
                for j in range(D // 16):
                    acc_m[r, pl.ds(j * 16, 16)] = zero16
                if mode == "gat":
                    acc_s[r, :] = zero16
                return 0

            lax.fori_loop(0, CHUNK, zero_row, 0)

            s0 = e0 // SB
            s1 = (e1 + (SB - 1)) // SB

            def sb_body(sb, _):
                ebase = pl.multiple_of(sb * SB, SB)
                pltpu.sync_copy(srcs_hbm.at[pl.ds(ebase, SB)], idx_v)
                pltpu.sync_copy(dsts_hbm.at[pl.ds(ebase, SB)],
                                dst_v.at[pl.ds(0, SB)])
                if mode == "gat":
                    for j in range(SB // 16):
                        sv = idx_v[pl.ds(j * 16, 16)]
                        dv = dst_v[pl.ds(j * 16, 16)]
                        logit = (plsc.load_gather(tabs_v[0], [sv])
                                 + plsc.load_gather(tabs_v[1], [dv]))
                        logit = jnp.where(logit > 0, logit, 0.2 * logit)
                        w_v[pl.ds(j * 16, 16)] = jnp.exp(logit)

                def acc_tile(k):
                    eb = ebase + k * KT
                    lo = jnp.maximum(e0 - eb, 0)
                    hi = jnp.minimum(e1 - eb, KT)
                    buf = k % 2

                    def edge_body(jj, _):
                        kk = k * KT + jj
                        ld = dst_v[pl.ds(kk, 16)][0] - base
                        if mode == "sage":
                            for j in range(D // 16):
                                plsc.addupdate(
                                    acc_m.at[ld, pl.ds(j * 16, 16)],
                                    rows_v[buf, jj, pl.ds(j * 16, 16)])
                        else:
                            wb = jnp.broadcast_to(w_v[pl.ds(kk, 16)][0],
                                                  (16,))
                            for j in range(D // 16):
                                plsc.addupdate(
                                    acc_m.at[ld, pl.ds(j * 16, 16)],
                                    wb * rows_v[buf, jj, pl.ds(j * 16, 16)])
                            if mode == "gat":
                                plsc.addupdate(acc_s.at[ld, :], wb)
                        return 0

                    lax.fori_loop(lo, hi, edge_body, 0)

                descs = [None] * TPS
                valids = [None] * TPS
                for k in range(TPS):
                    eb = ebase + k * KT
                    valids[k] = jnp.logical_and(eb < e1, eb + KT > e0)
                    descs[k] = pltpu.make_async_copy(
                        tab_hbm.at[idx_v.at[pl.ds(k * KT, KT)]],
                        rows_v.at[k % 2], sems[k % 2])

                    @pl.when(valids[k])
                    def _(k=k):
                        descs[k].start()

                    if k > 0:
                        @pl.when(valids[k - 1])
                        def _(k=k):
                            descs[k - 1].wait()
                            acc_tile(k - 1)

                @pl.when(valids[TPS - 1])
                def _():
                    descs[TPS - 1].wait()
                    acc_tile(TPS - 1)
                return 0

            lax.fori_loop(s0, s1, sb_body, 0)

            pltpu.sync_copy(acc_m, m_hbm.at[pl.ds(base, CHUNK)])
            if mode == "gat":
                pltpu.sync_copy(acc_s, s_hbm.at[pl.ds(base, CHUNK)])
            return 0

        lax.fori_loop(0, CPW, chunk_body, 0)

    return spmm


_SPMM_CACHE = {}


def _get_spmm(D, mode):
    key = (D, mode)
    if key not in _SPMM_CACHE:
        _SPMM_CACHE[key] = _make_spmm(D, mode)
    return _SPMM_CACHE[key]


_RBLK = 1024
_GRID = NP // _RBLK


def _row_spec(width):
    if width is None:
        return pl.BlockSpec((_RBLK,), lambda i: (i,))
    return pl.BlockSpec((_RBLK, width), lambda i: (i, 0))


def _full_spec(*shape):
    if len(shape) == 1:
        return pl.BlockSpec(shape, lambda i: (0,))
    return pl.BlockSpec(shape, lambda i: (0, 0))


def _ln(v, g, b):
    m = jnp.mean(v, axis=-1, keepdims=True)
    var = jnp.mean((v - m) ** 2, axis=-1, keepdims=True)
    return (v - m) * jax.lax.rsqrt(var + 1e-5) * g + b


def _elu(v):
    return jnp.where(v > 0, v, jnp.exp(jnp.minimum(v, 0.0)) - 1.0)


def _mm(a, b):
    return jnp.dot(a, b, preferred_element_type=jnp.float32)


def _p1_body(x_ref, w1_ref, a1s_ref, a1d_ref, h_ref, asc_ref, adc_ref):
    h = _mm(x_ref[...], w1_ref[...])
    h_ref[...] = h
    asc_ref[...] = _mm(h, a1s_ref[...])
    adc_ref[...] = _mm(h, a1d_ref[...])


def _p1(xp, W1, a1s, a1d):
    return pl.pallas_call(
        _p1_body,
        grid=(_GRID,),
        in_specs=[_row_spec(F_IN), _full_spec(F_IN, H), _full_spec(H),
                  _full_spec(H)],
        out_specs=[_row_spec(H), _row_spec(None), _row_spec(None)],
        out_shape=[_f32((NP, H)), _f32((NP,)), _f32((NP,))],
    )(xp, W1, a1s, a1d)


def _gat_combine(m, s, h, asc, adc, b):
    logit = asc + adc
    wself = jnp.exp(jnp.where(logit > 0, logit, 0.2 * logit))
    num = m + wself[:, None] * h
    den = s[:, 0:1] + wself[:, None] + 1e-16
    return num / den + b


def _e1_body(m_ref, s_ref, h_ref, asc_ref, adc_ref, cnt_ref, w2_ref,
             b1_ref, g1_ref, be1_ref, h2_ref, dinv_ref):
    gat = _gat_combine(m_ref[...], s_ref[...], h_ref[...], asc_ref[...],
                       adc_ref[...], b1_ref[...])
    act = _ln(_elu(gat), g1_ref[...], be1_ref[...])
    dinv = jax.lax.rsqrt(cnt_ref[...] + 1.0)
    h2_ref[...] = dinv[:, None] * _mm(act, w2_ref[...])
    dinv_ref[...] = dinv


def _e1(M1, S1, h1, asc1, adc1, cntf, W2, b1, g1, be1):
    return pl.pallas_call(
        _e1_body,
        grid=(_GRID,),
        in_specs=[_row_spec(H), _row_spec(16), _row_spec(H), _row_spec(None),
                  _row_spec(None), _row_spec(None), _full_spec(H, H),
                  _full_spec(H), _full_spec(H), _full_spec(H)],
        out_specs=[_row_spec(H), _row_spec(None)],
        out_shape=[_f32((NP, H)), _f32((NP,))],
    )(M1, S1, h1, asc1, adc1, cntf, W2, b1, g1, be1)


def _e2_body(m_ref, h2_ref, dinv_ref, b2_ref, g2_ref, be2_ref, wr3_ref,
             h3_ref, r3_ref):
    dinv = dinv_ref[...][:, None]
    gcn = dinv * (m_ref[...] + h2_ref[...]) + b2_ref[...]
    h3 = _ln(_elu(gcn), g2_ref[...], be2_ref[...])
    h3_ref[...] = h3
    r3_ref[...] = _mm(h3, wr3_ref[...])


def _e2(M2, h2, dinv, b2, g2, be2, Wr3):
    return pl.pallas_call(
        _e2_body,
        grid=(_GRID,),
        in_specs=[_row_spec(H), _row_spec(H), _row_spec(None), _full_spec(H),
                  _full_spec(H), _full_spec(H), _full_spec(H, H)],
        out_specs=[_row_spec(H), _row_spec(H)],
        out_shape=[_f32((NP, H)), _f32((NP, H))],
    )(M2, h2, dinv, b2, g2, be2, Wr3)


def _e3_body(m_ref, r3_ref, cnt_ref, wl3_ref, bl3_ref, g3_ref, be3_ref,
             w4_ref, a4s_ref, a4d_ref, x_ref, wres_ref, bres_ref,
             inter_ref, h4_ref, asc_ref, adc_ref, res_ref):
    mean = m_ref[...] / jnp.maximum(cnt_ref[...], 1.0)[:, None]
    sage = _mm(mean, wl3_ref[...]) + bl3_ref[...] + r3_ref[...]
    inter = _ln(_elu(sage), g3_ref[...], be3_ref[...])
    inter_ref[...] = inter
    h4 = _mm(inter, w4_ref[...])
    h4_ref[...] = h4
    asc_ref[...] = _mm(h4, a4s_ref[...])
    adc_ref[...] = _mm(h4, a4d_ref[...])
    res_ref[...] = _mm(x_ref[...], wres_ref[...]) + bres_ref[...]


def _e3(M3, r3, cntf, Wl3, bl3, g3, be3, W4, a4s, a4d, xp, Wres, bres):
    return pl.pallas_call(
        _e3_body,
        grid=(_GRID,),
        in_specs=[_row_spec(H), _row_spec(H), _row_spec(None),
                  _full_spec(H, H), _full_spec(H), _full_spec(H),
                  _full_spec(H), _full_spec(H, C), _full_spec(C),
                  _full_spec(C), _row_spec(F_IN), _full_spec(F_IN, C),
                  _full_spec(C)],
        out_specs=[_row_spec(H), _row_spec(C), _row_spec(None),
                   _row_spec(None), _row_spec(C)],
        out_shape=[_f32((NP, H)), _f32((NP, C)), _f32((NP,)), _f32((NP,)),
                   _f32((NP, C))],
    )(M3, r3, cntf, Wl3, bl3, g3, be3, W4, a4s, a4d, xp, Wres, bres)


def _e4_body(m_ref, s_ref, h4_ref, asc_ref, adc_ref, b4_ref, res_ref,
             out_ref):
    out_ref[...] = _gat_combine(m_ref[...], s_ref[...], h4_ref[...],
                                asc_ref[...], adc_ref[...],
                                b4_ref[...]) + res_ref[...]


def _e4(M4, S4, h4, asc4, adc4, b4, res):
    return pl.pallas_call(
        _e4_body,
        grid=(_GRID,),
        in_specs=[_row_spec(C), _row_spec(16), _row_spec(C), _row_spec(None),
                  _row_spec(None), _full_spec(C), _row_spec(C)],
        out_specs=_row_spec(C),
        out_shape=_f32((NP, C)),
    )(M4, S4, h4, asc4, adc4, b4, res)


def kernel(x, edge_index, W1, a1s, a1d, b1, g1, be1, W2, b2, g2, be2,
           Wl3, bl3, Wr3, g3, be3, W4, a4s, a4d, b4, Wres, bres):
    src = edge_index[0]
    dst = edge_index[1]
    eidx = jnp.arange(E, dtype=jnp.uint32)
    key = jnp.sort(dst.astype(jnp.uint32) * jnp.uint32(1 << 18) + eidx)
    dst_s = (key >> 18).astype(jnp.int32)
    order = (key & jnp.uint32((1 << 18) - 1)).astype(jnp.int32)
    src_s = jnp.take(src, order).astype(jnp.int32)
    ptr = jnp.searchsorted(
        dst_s, jnp.arange(NP + 1, dtype=jnp.int32)).astype(jnp.int32)
    src_s = jnp.zeros((EP,), jnp.int32).at[:E].set(src_s)
    dst_s = jnp.zeros((EP,), jnp.int32).at[:E].set(dst_s)
    cb = jnp.zeros((NCHUNK + 16,), jnp.int32)
    cb = cb.at[: NCHUNK + 1].set(ptr[:: CHUNK])
    cntf = (ptr[1:] - ptr[:-1]).astype(jnp.float32)

    xp = jnp.zeros((NP, F_IN), jnp.float32).at[:N].set(x)

    h1, asc1, adc1 = _p1(xp, W1, a1s, a1d)
    M1, S1 = _get_spmm(H, "gat")(h1, src_s, dst_s, cb, asc1, adc1)
    h2, dinv = _e1(M1, S1, h1, asc1, adc1, cntf, W2, b1, g1, be1)

    M2, = _get_spmm(H, "sage")(h2, src_s, dst_s, cb)
    h3, r3 = _e2(M2, h2, dinv, b2, g2, be2, Wr3)

    M3, = _get_spmm(H, "sage")(h3, src_s, dst_s, cb)
    inter, h4, asc4, adc4, res = _e3(M3, r3, cntf, Wl3, bl3, g3, be3,
                                     W4, a4s, a4d, xp, Wres, bres)

    M4, S4 = _get_spmm(C, "gat")(h4, src_s, dst_s, cb, asc4, adc4)
    out = _e4(M4, S4, h4, asc4, adc4, b4, res)

    return inter[:N], out[:N]
